# Initial kernel scaffold; baseline (speedup 1.0000x reference)
import jax
import jax.numpy as jnp
from jax.experimental import pallas as pl


def kernel(X, R, batch, W_emb, b_emb, Wr, br, Wp, bp, Wa1, ba1, Wa2, ba2, Wo1, bo1, Wo2, bo2):
    raise NotImplementedError("write your pallas kernel here")


# speedup vs baseline: 16.7523x; 16.7523x over previous
import jax
import jax.numpy as jnp
from jax.experimental import pallas as pl
from jax.experimental.pallas import tpu as pltpu

_A = 64
_F = 32
_H = 64
_NRBF = 64
_GAMMA = 10.0
_CUTOFF = 10.0
_NI = 3
_MPP = 2


def _silu(x):
    return x * jax.nn.sigmoid(x)


def _pair_dist(Ri):
    prod = jax.lax.dot_general(Ri, Ri, (((1,), (1,)), ((), ())),
                               preferred_element_type=jnp.float32)
    R2 = Ri * Ri
    sq_col = jnp.sum(R2, axis=1, keepdims=True)
    ones_row = jnp.ones((1, 3), jnp.float32)
    sq_row = jax.lax.dot_general(ones_row, R2, (((1,), (1,)), ((), ())),
                                 preferred_element_type=jnp.float32)
    d2 = sq_col + sq_row - 2.0 * prod
    row_i = jax.lax.broadcasted_iota(jnp.int32, (_A, _A), 0)
    col_i = jax.lax.broadcasted_iota(jnp.int32, (_A, _A), 1)
    d2 = jnp.where(row_i == col_i, 0.0, d2)
    return jnp.sqrt(jnp.maximum(d2, 1e-12))


def _schnet_kernel(X_ref, R_ref, We_ref, be_ref, Wr_ref, br_ref, Wp_ref,
                   bp_ref, Wa1_ref, ba1_ref, Wa2_ref, ba2_ref, Wo1_ref,
                   bo1_ref, Wo2t_ref, bo2_ref, y_ref):
    lane = jax.lax.broadcasted_iota(jnp.int32, (1, 1, 2 * _NRBF), 2)
    step = _CUTOFF / (_NRBF - 1)
    centers2 = (lane % _NRBF).astype(jnp.float32) * step
    in_first = lane < _NRBF

    d_a = _pair_dist(R_ref[0])
    d_b = _pair_dist(R_ref[1])
    dsel = jnp.where(in_first, d_a[:, :, None], d_b[:, :, None])
    delta = dsel - centers2
    rbf = jnp.exp(-_GAMMA * delta * delta)
    Spack = jnp.sum(rbf, axis=0)
    S = jnp.concatenate([Spack[:, :_NRBF], Spack[:, _NRBF:]], axis=0)

    X2 = X_ref[...].reshape(_MPP * _A, _F)
    h = jnp.dot(X2, We_ref[...], preferred_element_type=jnp.float32) + be_ref[...]
    nA = float(_A)
    for i in range(_NI):
        e = jnp.dot(S, Wr_ref[i], preferred_element_type=jnp.float32) \
            + nA * br_ref[i:i + 1, :]
        agg = jnp.dot(e, Wp_ref[i], preferred_element_type=jnp.float32) \
            + nA * bp_ref[i:i + 1, :]
        pre = jnp.dot(agg, Wa1_ref[i], preferred_element_type=jnp.float32) \
            + ba1_ref[i:i + 1, :]
        t = jnp.dot(_silu(pre), Wa2_ref[i], preferred_element_type=jnp.float32) \
            + ba2_ref[i:i + 1, :]
        h = h + t

    u = _silu(jnp.dot(h, Wo1_ref[...], preferred_element_type=jnp.float32)
              + bo1_ref[...])
    o = jax.lax.dot_general(u, Wo2t_ref[...], (((1,), (1,)), ((), ())),
                            preferred_element_type=jnp.float32)
    ya = jnp.sum(o[0:_A, :], axis=0, keepdims=True) / nA + bo2_ref[...]
    yb = jnp.sum(o[_A:, :], axis=0, keepdims=True) / nA + bo2_ref[...]
    y_ref[0] = jnp.concatenate([ya, yb], axis=0)


def kernel(X, R, batch, W_emb, b_emb, Wr, br, Wp, bp, Wa1, ba1, Wa2, ba2,
           Wo1, bo1, Wo2, bo2):
    Bn = X.shape[0]
    grid = (Bn // _MPP,)
    be = b_emb.reshape(1, _H)
    bo1r = bo1.reshape(1, _H)
    bo2r = bo2.reshape(1, 1)
    Wo2t = Wo2.reshape(1, _H)

    full2 = lambda b: (0, 0)
    full3 = lambda b: (0, 0, 0)
    out = pl.pallas_call(
        _schnet_kernel,
        grid=grid,
        in_specs=[
            pl.BlockSpec((_MPP, _A, _F), lambda b: (b, 0, 0)),
            pl.BlockSpec((_MPP, _A, 3), lambda b: (b, 0, 0)),
            pl.BlockSpec((_F, _H), full2),
            pl.BlockSpec((1, _H), full2),
            pl.BlockSpec((_NI, _NRBF, _H), full3),
            pl.BlockSpec((_NI, _H), full2),
            pl.BlockSpec((_NI, _H, _H), full3),
            pl.BlockSpec((_NI, _H), full2),
            pl.BlockSpec((_NI, _H, _H), full3),
            pl.BlockSpec((_NI, _H), full2),
            pl.BlockSpec((_NI, _H, _H), full3),
            pl.BlockSpec((_NI, _H), full2),
            pl.BlockSpec((_H, _H), full2),
            pl.BlockSpec((1, _H), full2),
            pl.BlockSpec((1, _H), full2),
            pl.BlockSpec((1, 1), full2),
        ],
        out_specs=pl.BlockSpec((1, _MPP, 1), lambda b: (b, 0, 0)),
        out_shape=jax.ShapeDtypeStruct((Bn // _MPP, _MPP, 1), jnp.float32),
        compiler_params=pltpu.CompilerParams(
            dimension_semantics=("arbitrary",)),
    )(X, R, W_emb, be, Wr, br, Wp, bp, Wa1, ba1, Wa2, ba2, Wo1, bo1r,
      Wo2t, bo2r)
    return out.reshape(Bn, 1)


# speedup vs baseline: 21.3227x; 1.2728x over previous
import jax
import jax.numpy as jnp
from jax.experimental import pallas as pl
from jax.experimental.pallas import tpu as pltpu

_A = 64
_F = 32
_H = 64
_NRBF = 64
_GAMMA = 10.0
_CUTOFF = 10.0
_NI = 3
_MPP = 2


def _silu(x):
    return x * jax.nn.sigmoid(x)


def _pair_dist(Ri):
    prod = jax.lax.dot_general(Ri, Ri, (((1,), (1,)), ((), ())),
                               preferred_element_type=jnp.float32)
    R2 = Ri * Ri
    sq_col = jnp.sum(R2, axis=1, keepdims=True)
    ones_row = jnp.ones((1, 3), jnp.float32)
    sq_row = jax.lax.dot_general(ones_row, R2, (((1,), (1,)), ((), ())),
                                 preferred_element_type=jnp.float32)
    d2 = sq_col + sq_row - 2.0 * prod
    row_i = jax.lax.broadcasted_iota(jnp.int32, (_A, _A), 0)
    col_i = jax.lax.broadcasted_iota(jnp.int32, (_A, _A), 1)
    d2 = jnp.where(row_i == col_i, 0.0, d2)
    return jnp.sqrt(jnp.maximum(d2, 1e-12))


def _schnet_kernel(X_ref, R_ref, We_ref, be_ref, Wr_ref, br_ref, Wp_ref,
                   bp_ref, Wa1_ref, ba1_ref, Wa2_ref, ba2_ref, Wo1_ref,
                   bo1_ref, Wo2t_ref, bo2_ref, y_ref):
    step = _CUTOFF / (_NRBF - 1)
    tscale = (_GAMMA * 1.4426950408889634) ** 0.5
    cgrid = jax.lax.broadcasted_iota(
        jnp.int32, (_NRBF, _MPP * _A), 0).astype(jnp.float32) * (step * tscale)

    d_a = _pair_dist(R_ref[0])
    d_b = _pair_dist(R_ref[1])
    ds = jnp.concatenate([d_a, d_b], axis=1) * tscale
    T = ds[:, None, :] - cgrid[None, :, :]
    N = cgrid[None, :, :] - ds[:, None, :]
    rbf = jnp.exp2(T * N)
    Srq = jnp.sum(rbf, axis=0)

    X2 = X_ref[...].reshape(_MPP * _A, _F)
    h = jnp.dot(X2, We_ref[...], preferred_element_type=jnp.float32) + be_ref[...]
    nA = float(_A)
    for i in range(_NI):
        e = jax.lax.dot_general(Srq, Wr_ref[i], (((0,), (0,)), ((), ())),
                                preferred_element_type=jnp.float32) \
            + nA * br_ref[i:i + 1, :]
        agg = jnp.dot(e, Wp_ref[i], preferred_element_type=jnp.float32) \
            + nA * bp_ref[i:i + 1, :]
        pre = jnp.dot(agg, Wa1_ref[i], preferred_element_type=jnp.float32) \
            + ba1_ref[i:i + 1, :]
        t = jnp.dot(_silu(pre), Wa2_ref[i], preferred_element_type=jnp.float32) \
            + ba2_ref[i:i + 1, :]
        h = h + t

    u = _silu(jnp.dot(h, Wo1_ref[...], preferred_element_type=jnp.float32)
              + bo1_ref[...])
    o = jax.lax.dot_general(u, Wo2t_ref[...], (((1,), (1,)), ((), ())),
                            preferred_element_type=jnp.float32)
    ya = jnp.sum(o[0:_A, :], axis=0, keepdims=True) / nA + bo2_ref[...]
    yb = jnp.sum(o[_A:, :], axis=0, keepdims=True) / nA + bo2_ref[...]
    y_ref[0] = jnp.concatenate([ya, yb], axis=0)


def kernel(X, R, batch, W_emb, b_emb, Wr, br, Wp, bp, Wa1, ba1, Wa2, ba2,
           Wo1, bo1, Wo2, bo2):
    Bn = X.shape[0]
    grid = (Bn // _MPP,)
    be = b_emb.reshape(1, _H)
    bo1r = bo1.reshape(1, _H)
    bo2r = bo2.reshape(1, 1)
    Wo2t = Wo2.reshape(1, _H)

    full2 = lambda b: (0, 0)
    full3 = lambda b: (0, 0, 0)
    out = pl.pallas_call(
        _schnet_kernel,
        grid=grid,
        in_specs=[
            pl.BlockSpec((_MPP, _A, _F), lambda b: (b, 0, 0)),
            pl.BlockSpec((_MPP, _A, 3), lambda b: (b, 0, 0)),
            pl.BlockSpec((_F, _H), full2),
            pl.BlockSpec((1, _H), full2),
            pl.BlockSpec((_NI, _NRBF, _H), full3),
            pl.BlockSpec((_NI, _H), full2),
            pl.BlockSpec((_NI, _H, _H), full3),
            pl.BlockSpec((_NI, _H), full2),
            pl.BlockSpec((_NI, _H, _H), full3),
            pl.BlockSpec((_NI, _H), full2),
            pl.BlockSpec((_NI, _H, _H), full3),
            pl.BlockSpec((_NI, _H), full2),
            pl.BlockSpec((_H, _H), full2),
            pl.BlockSpec((1, _H), full2),
            pl.BlockSpec((1, _H), full2),
            pl.BlockSpec((1, 1), full2),
        ],
        out_specs=pl.BlockSpec((1, _MPP, 1), lambda b: (b, 0, 0)),
        out_shape=jax.ShapeDtypeStruct((Bn // _MPP, _MPP, 1), jnp.float32),
        compiler_params=pltpu.CompilerParams(
            dimension_semantics=("arbitrary",)),
    )(X, R, W_emb, be, Wr, br, Wp, bp, Wa1, ba1, Wa2, ba2, Wo1, bo1r,
      Wo2t, bo2r)
    return out.reshape(Bn, 1)
